# baseline (speedup 1.0000x reference)
import jax
import jax.numpy as jnp
from jax import lax
from jax.experimental import pallas as pl
from jax.experimental.pallas import tpu as pltpu
from jax.experimental.pallas import tpu_sc as plsc

N_CLAUSES = 10000
N_VARS = 10000
E = 320000
D = 128

NC = 2
NS = 16
NW = NC * NS
K = 96
EPW = 10080
E_PAD = NW * EPW
NCHUNK = EPW // K
NV_PAD = 10112
RPT = NV_PAD // NS


NBUF = 2
NGROUP = NCHUNK // NBUF


def _seg_sum_sc(src1, dst3, xc, zeros, out, srcidx_v, dstidx_v,
                r0, r1, acc_sh, s0, s1):
    cid = lax.axis_index("c")
    sid = lax.axis_index("s")
    wid = sid * NC + cid
    rows = (r0, r1)
    sems = (s0, s1)

    pltpu.sync_copy(src1.at[pl.ds(wid * EPW, EPW)], srcidx_v)
    pltpu.sync_copy(dst3.at[wid], dstidx_v)

    for b in range(NBUF):
        pltpu.async_copy(xc.at[srcidx_v.at[pl.ds(b * K, K)]], rows[b],
                         sems[b])
    pltpu.sync_copy(zeros, acc_sh.at[pl.ds(sid * RPT, RPT)])
    plsc.subcore_barrier()

    def group(g, issue_next):
        for b in range(NBUF):
            j = g * NBUF + b
            pltpu.make_async_copy(xc.at[srcidx_v.at[pl.ds(j * K, K)]],
                                  rows[b], sems[b]).wait()
            pltpu.sync_copy(rows[b], acc_sh.at[dstidx_v.at[j]], add=True)
            if issue_next:
                pltpu.async_copy(
                    xc.at[srcidx_v.at[pl.ds((j + NBUF) * K, K)]],
                    rows[b], sems[b])

    def body(i, carry):
        group(2 * i, True)
        group(2 * i + 1, True)
        return carry

    lax.fori_loop(0, (NGROUP - 2) // 2, body, 0)
    group(NGROUP - 2, True)
    group(NGROUP - 1, False)

    for j in range(NGROUP * NBUF, NCHUNK):
        pltpu.async_copy(xc.at[srcidx_v.at[pl.ds(j * K, K)]], rows[0],
                         sems[0]).wait()
        pltpu.sync_copy(rows[0], acc_sh.at[dstidx_v.at[j]], add=True)

    plsc.subcore_barrier()
    pltpu.sync_copy(acc_sh.at[pl.ds(sid * RPT, RPT)],
                    out.at[cid, pl.ds(sid * RPT, RPT)])


def _segment_sum(src1, dst3, xc, zeros):
    mesh = plsc.VectorSubcoreMesh(core_axis_name="c", subcore_axis_name="s")
    f = pl.kernel(
        _seg_sum_sc,
        out_type=jax.ShapeDtypeStruct((2, NV_PAD, D), jnp.float32),
        mesh=mesh,
        scratch_types=[
            pltpu.VMEM((EPW,), jnp.int32),
            pltpu.VMEM((NCHUNK, K), jnp.int32),
        ] + [pltpu.VMEM((K, D), jnp.float32) for _ in range(NBUF)] + [
            pltpu.VMEM_SHARED((NV_PAD, D), jnp.float32),
        ] + [pltpu.SemaphoreType.DMA for _ in range(NBUF)],
    )
    return f(src1, dst3, xc, zeros)


BLK = 1000
NBLK = N_VARS // BLK


def _lstm_tc(ma_ref, mb_ref, h_ref, c_ref, wih_ref, whh_ref, b_ref,
             ho_ref, co_ref):
    msg = ma_ref[0] + mb_ref[0]
    gates = (jnp.dot(msg, wih_ref[...], preferred_element_type=jnp.float32)
             + jnp.dot(h_ref[...], whh_ref[...],
                       preferred_element_type=jnp.float32)
             + b_ref[...])
    ii = jax.nn.sigmoid(gates[:, 0:D])
    ff = jax.nn.sigmoid(gates[:, D:2 * D])
    gg = jnp.tanh(gates[:, 2 * D:3 * D])
    oo = jax.nn.sigmoid(gates[:, 3 * D:4 * D])
    c_new = ff * c_ref[...] + ii * gg
    ho_ref[...] = oo * jnp.tanh(c_new)
    co_ref[...] = c_new


def _lstm(msg2, h, c, wih_t, whh_t, b2):
    row_spec = pl.BlockSpec((BLK, D), lambda i: (i, 0))
    return pl.pallas_call(
        _lstm_tc,
        grid=(NBLK,),
        in_specs=[
            pl.BlockSpec((1, BLK, D), lambda i: (0, i, 0)),
            pl.BlockSpec((1, BLK, D), lambda i: (1, i, 0)),
            row_spec,
            row_spec,
            pl.BlockSpec((D, 4 * D), lambda i: (0, 0)),
            pl.BlockSpec((D, 4 * D), lambda i: (0, 0)),
            pl.BlockSpec((1, 4 * D), lambda i: (0, 0)),
        ],
        out_specs=[row_spec, row_spec],
        out_shape=[
            jax.ShapeDtypeStruct((N_VARS, D), jnp.float32),
            jax.ShapeDtypeStruct((N_VARS, D), jnp.float32),
        ],
    )(msg2, msg2, h, c, wih_t, whh_t, b2)


def kernel(edge_index, x_c, h, c, v_batch, W_ih, W_hh, b_ih, b_hh):
    npad = E_PAD - E
    src1 = jnp.concatenate(
        [edge_index[0], jnp.zeros((npad,), jnp.int32)])
    dst_dummy = N_VARS + (jnp.arange(npad, dtype=jnp.int32)
                          % (NV_PAD - N_VARS))
    dst3 = jnp.concatenate([edge_index[1], dst_dummy]).reshape(
        NW, NCHUNK, K)
    zeros = jnp.zeros((RPT, D), jnp.float32)
    msg2 = _segment_sum(src1, dst3, x_c, zeros)
    wih_t = W_ih.T
    whh_t = W_hh.T
    b2 = (b_ih + b_hh).reshape(1, 4 * D)
    h_new, c_new = _lstm(msg2, h, c, wih_t, whh_t, b2)
    return (h_new, c_new)


# speedup vs baseline: 1.4743x; 1.4743x over previous
import jax
import jax.numpy as jnp
from jax import lax
from jax.experimental import pallas as pl
from jax.experimental.pallas import tpu as pltpu
from jax.experimental.pallas import tpu_sc as plsc

N_CLAUSES = 10000
N_VARS = 10000
E = 320000
D = 128

NC = 2
NS = 16
NW = NC * NS
EPW = E // NW
K = 80
NCHUNK = EPW // K
NV_PAD = 10240
RPT = NV_PAD // NS


NBUF = 2
NGROUP = NCHUNK // NBUF


def _seg_sum_sc(src1, dst3, xc, zeros, out, srcidx_v, dstidx_v,
                r0, r1, acc_sh, s0, s1):
    cid = lax.axis_index("c")
    sid = lax.axis_index("s")
    wid = sid * NC + cid
    rows = (r0, r1)
    sems = (s0, s1)

    pltpu.sync_copy(src1.at[pl.ds(wid * EPW, EPW)], srcidx_v)
    pltpu.sync_copy(dst3.at[wid], dstidx_v)

    for b in range(NBUF):
        pltpu.async_copy(xc.at[srcidx_v.at[pl.ds(b * K, K)]], rows[b],
                         sems[b])
    pltpu.sync_copy(zeros, acc_sh.at[pl.ds(sid * RPT, RPT)])
    plsc.subcore_barrier()

    def group(g, issue_next):
        for b in range(NBUF):
            j = g * NBUF + b
            pltpu.make_async_copy(xc.at[srcidx_v.at[pl.ds(j * K, K)]],
                                  rows[b], sems[b]).wait()
            pltpu.sync_copy(rows[b], acc_sh.at[dstidx_v.at[j]], add=True)
            if issue_next:
                pltpu.async_copy(
                    xc.at[srcidx_v.at[pl.ds((j + NBUF) * K, K)]],
                    rows[b], sems[b])

    def body(i, carry):
        group(2 * i, True)
        group(2 * i + 1, True)
        return carry

    lax.fori_loop(0, (NGROUP - 2) // 2, body, 0)
    group(NGROUP - 2, True)
    group(NGROUP - 1, False)

    for j in range(NGROUP * NBUF, NCHUNK):
        pltpu.async_copy(xc.at[srcidx_v.at[pl.ds(j * K, K)]], rows[0],
                         sems[0]).wait()
        pltpu.sync_copy(rows[0], acc_sh.at[dstidx_v.at[j]], add=True)

    plsc.subcore_barrier()
    pltpu.sync_copy(acc_sh.at[pl.ds(sid * RPT, RPT)],
                    out.at[cid, pl.ds(sid * RPT, RPT)])


def _segment_sum(src1, dst3, xc, zeros):
    mesh = plsc.VectorSubcoreMesh(core_axis_name="c", subcore_axis_name="s")
    f = pl.kernel(
        _seg_sum_sc,
        out_type=jax.ShapeDtypeStruct((2, NV_PAD, D), jnp.float32),
        mesh=mesh,
        scratch_types=[
            pltpu.VMEM((EPW,), jnp.int32),
            pltpu.VMEM((NCHUNK, K), jnp.int32),
        ] + [pltpu.VMEM((K, D), jnp.float32) for _ in range(NBUF)] + [
            pltpu.VMEM_SHARED((NV_PAD, D), jnp.float32),
        ] + [pltpu.SemaphoreType.DMA for _ in range(NBUF)],
    )
    return f(src1, dst3, xc, zeros)


BLK = 1000
NBLK = N_VARS // BLK


def _lstm_tc(ma_ref, mb_ref, h_ref, c_ref, wih_ref, whh_ref, b_ref,
             ho_ref, co_ref):
    msg = ma_ref[0] + mb_ref[0]
    gates = (jnp.dot(msg, wih_ref[...], preferred_element_type=jnp.float32)
             + jnp.dot(h_ref[...], whh_ref[...],
                       preferred_element_type=jnp.float32)
             + b_ref[...])
    ii = jax.nn.sigmoid(gates[:, 0:D])
    ff = jax.nn.sigmoid(gates[:, D:2 * D])
    gg = jnp.tanh(gates[:, 2 * D:3 * D])
    oo = jax.nn.sigmoid(gates[:, 3 * D:4 * D])
    c_new = ff * c_ref[...] + ii * gg
    ho_ref[...] = oo * jnp.tanh(c_new)
    co_ref[...] = c_new


def _lstm(msg2, h, c, wih_t, whh_t, b2):
    row_spec = pl.BlockSpec((BLK, D), lambda i: (i, 0))
    return pl.pallas_call(
        _lstm_tc,
        grid=(NBLK,),
        in_specs=[
            pl.BlockSpec((1, BLK, D), lambda i: (0, i, 0)),
            pl.BlockSpec((1, BLK, D), lambda i: (1, i, 0)),
            row_spec,
            row_spec,
            pl.BlockSpec((D, 4 * D), lambda i: (0, 0)),
            pl.BlockSpec((D, 4 * D), lambda i: (0, 0)),
            pl.BlockSpec((1, 4 * D), lambda i: (0, 0)),
        ],
        out_specs=[row_spec, row_spec],
        out_shape=[
            jax.ShapeDtypeStruct((N_VARS, D), jnp.float32),
            jax.ShapeDtypeStruct((N_VARS, D), jnp.float32),
        ],
    )(msg2, msg2, h, c, wih_t, whh_t, b2)


def kernel(edge_index, x_c, h, c, v_batch, W_ih, W_hh, b_ih, b_hh):
    src1 = edge_index[0]
    dst3 = edge_index[1].reshape(NW, NCHUNK, K)
    zeros = jnp.zeros((RPT, D), jnp.float32)
    msg2 = _segment_sum(src1, dst3, x_c, zeros)
    wih_t = W_ih.T
    whh_t = W_hh.T
    b2 = (b_ih + b_hh).reshape(1, 4 * D)
    h_new, c_new = _lstm(msg2, h, c, wih_t, whh_t, b2)
    return (h_new, c_new)


# speedup vs baseline: 1.4847x; 1.0071x over previous
import jax
import jax.numpy as jnp
from jax import lax
from jax.experimental import pallas as pl
from jax.experimental.pallas import tpu as pltpu
from jax.experimental.pallas import tpu_sc as plsc

N_CLAUSES = 10000
N_VARS = 10000
E = 320000
D = 128

NC = 2
NS = 16
NW = NC * NS
EPW = E // NW
K = 80
NCHUNK = EPW // K
NV_PAD = 10240
RPT = NV_PAD // NS


NBUF = 2
NGROUP = NCHUNK // NBUF


def _seg_sum_sc(src1, dst3, xc, zeros, out, srcidx_v, dstidx_v,
                r0, r1, acc_sh, s0, s1):
    cid = lax.axis_index("c")
    sid = lax.axis_index("s")
    wid = sid * NC + cid
    rows = (r0, r1)
    sems = (s0, s1)

    pltpu.sync_copy(src1.at[pl.ds(wid * EPW, EPW)], srcidx_v)
    pltpu.sync_copy(dst3.at[wid], dstidx_v)

    for b in range(NBUF):
        pltpu.async_copy(xc.at[srcidx_v.at[pl.ds(b * K, K)]], rows[b],
                         sems[b])
    pltpu.sync_copy(zeros, acc_sh.at[pl.ds(sid * RPT, RPT)])
    plsc.subcore_barrier()

    def group(g, issue_next):
        for b in range(NBUF):
            j = g * NBUF + b
            pltpu.make_async_copy(xc.at[srcidx_v.at[pl.ds(j * K, K)]],
                                  rows[b], sems[b]).wait()
            pltpu.sync_copy(rows[b], acc_sh.at[dstidx_v.at[j]], add=True)
            if issue_next:
                pltpu.async_copy(
                    xc.at[srcidx_v.at[pl.ds((j + NBUF) * K, K)]],
                    rows[b], sems[b])

    def body(i, carry):
        group(2 * i, True)
        group(2 * i + 1, True)
        return carry

    lax.fori_loop(0, (NGROUP - 2) // 2, body, 0)
    group(NGROUP - 2, True)
    group(NGROUP - 1, False)

    for j in range(NGROUP * NBUF, NCHUNK):
        pltpu.async_copy(xc.at[srcidx_v.at[pl.ds(j * K, K)]], rows[0],
                         sems[0]).wait()
        pltpu.sync_copy(rows[0], acc_sh.at[dstidx_v.at[j]], add=True)

    plsc.subcore_barrier()
    pltpu.sync_copy(acc_sh.at[pl.ds(sid * RPT, RPT)],
                    out.at[cid, pl.ds(sid * RPT, RPT)])


def _segment_sum(src1, dst3, xc, zeros):
    mesh = plsc.VectorSubcoreMesh(core_axis_name="c", subcore_axis_name="s")
    f = pl.kernel(
        _seg_sum_sc,
        out_type=jax.ShapeDtypeStruct((2, NV_PAD, D), jnp.float32),
        mesh=mesh,
        scratch_types=[
            pltpu.VMEM((EPW,), jnp.int32),
            pltpu.VMEM((NCHUNK, K), jnp.int32),
        ] + [pltpu.VMEM((K, D), jnp.float32) for _ in range(NBUF)] + [
            pltpu.VMEM_SHARED((NV_PAD, D), jnp.float32),
        ] + [pltpu.SemaphoreType.DMA for _ in range(NBUF)],
    )
    return f(src1, dst3, xc, zeros)


BLK = 2000
NBLK = N_VARS // BLK


def _lstm_tc(ma_ref, mb_ref, h_ref, c_ref, wih_ref, whh_ref, b_ref,
             ho_ref, co_ref):
    msg = ma_ref[0] + mb_ref[0]
    gates = (jnp.dot(msg, wih_ref[...], preferred_element_type=jnp.float32)
             + jnp.dot(h_ref[...], whh_ref[...],
                       preferred_element_type=jnp.float32)
             + b_ref[...])
    ii = jax.nn.sigmoid(gates[:, 0:D])
    ff = jax.nn.sigmoid(gates[:, D:2 * D])
    gg = jnp.tanh(gates[:, 2 * D:3 * D])
    oo = jax.nn.sigmoid(gates[:, 3 * D:4 * D])
    c_new = ff * c_ref[...] + ii * gg
    ho_ref[...] = oo * jnp.tanh(c_new)
    co_ref[...] = c_new


def _lstm(msg2, h, c, wih_t, whh_t, b2):
    row_spec = pl.BlockSpec((BLK, D), lambda i: (i, 0))
    return pl.pallas_call(
        _lstm_tc,
        grid=(NBLK,),
        in_specs=[
            pl.BlockSpec((1, BLK, D), lambda i: (0, i, 0)),
            pl.BlockSpec((1, BLK, D), lambda i: (1, i, 0)),
            row_spec,
            row_spec,
            pl.BlockSpec((D, 4 * D), lambda i: (0, 0)),
            pl.BlockSpec((D, 4 * D), lambda i: (0, 0)),
            pl.BlockSpec((1, 4 * D), lambda i: (0, 0)),
        ],
        out_specs=[row_spec, row_spec],
        out_shape=[
            jax.ShapeDtypeStruct((N_VARS, D), jnp.float32),
            jax.ShapeDtypeStruct((N_VARS, D), jnp.float32),
        ],
    )(msg2, msg2, h, c, wih_t, whh_t, b2)


def kernel(edge_index, x_c, h, c, v_batch, W_ih, W_hh, b_ih, b_hh):
    src1 = edge_index[0]
    dst3 = edge_index[1].reshape(NW, NCHUNK, K)
    zeros = jnp.zeros((RPT, D), jnp.float32)
    msg2 = _segment_sum(src1, dst3, x_c, zeros)
    wih_t = W_ih.T
    whh_t = W_hh.T
    b2 = (b_ih + b_hh).reshape(1, 4 * D)
    h_new, c_new = _lstm(msg2, h, c, wih_t, whh_t, b2)
    return (h_new, c_new)
